# Initial kernel scaffold; baseline (speedup 1.0000x reference)
import jax
import jax.numpy as jnp
from jax.experimental import pallas as pl


def kernel(x, train_pos_edge_index, W1, b1, W2, b2):
    raise NotImplementedError("write your pallas kernel here")


# speedup vs baseline: 7.5387x; 7.5387x over previous
import functools

import jax
import jax.numpy as jnp
from jax import lax
from jax.experimental import pallas as pl
from jax.experimental.pallas import tpu as pltpu
from jax.experimental.pallas import tpu_sc as plsc

N_NODES = 10000
N_EDGES = 320000
D = 128

NUM_CORES = 2
NUM_SUBCORES = 16
NUM_TILES = NUM_CORES * NUM_SUBCORES

CHUNK = 128
BC = 16
BLOCKS = 5
CHUNKS_PER_TILE = BC * BLOCKS
EDGES_PER_TILE = CHUNK * CHUNKS_PER_TILE
E_PAD = EDGES_PER_TILE * NUM_TILES
ACC_ROWS = 10112
ZROWS = ACC_ROWS // NUM_SUBCORES
DEG_W = 16

_mesh = plsc.VectorSubcoreMesh(core_axis_name="c", subcore_axis_name="s")


@functools.partial(
    pl.kernel,
    out_type=jax.ShapeDtypeStruct((NUM_CORES, ACC_ROWS, D), jnp.float32),
    mesh=_mesh,
    scratch_types=[
        pltpu.VMEM((CHUNKS_PER_TILE, CHUNK), jnp.int32),
        pltpu.VMEM((CHUNK, D), jnp.float32),
        pltpu.VMEM_SHARED((ACC_ROWS, D), jnp.float32),
    ],
)
def _sc_degree(dstp_hbm, zrows_hbm, ones_hbm, out_hbm, dst_v, ones_v, deg_sh):
    cid = lax.axis_index("c")
    sid = lax.axis_index("s")
    wid = cid * NUM_SUBCORES + sid
    zbase = sid * ZROWS
    pltpu.sync_copy(zrows_hbm, deg_sh.at[pl.ds(zbase, ZROWS)])
    pltpu.sync_copy(ones_hbm, ones_v)
    pltpu.sync_copy(dstp_hbm.at[wid], dst_v)
    plsc.subcore_barrier()

    def body(j, carry):
        pltpu.sync_copy(ones_v, deg_sh.at[dst_v.at[j]], add=True)
        return carry

    lax.fori_loop(0, CHUNKS_PER_TILE, body, 0)
    plsc.subcore_barrier()
    pltpu.sync_copy(deg_sh.at[pl.ds(zbase, ZROWS)],
                    out_hbm.at[cid, pl.ds(zbase, ZROWS)])


@functools.partial(
    pl.kernel,
    out_type=jax.ShapeDtypeStruct((NUM_CORES, ACC_ROWS, D), jnp.float32),
    mesh=_mesh,
    scratch_types=[
        pltpu.VMEM((CHUNKS_PER_TILE, CHUNK), jnp.int32),
        pltpu.VMEM((CHUNKS_PER_TILE, CHUNK), jnp.int32),
        pltpu.VMEM((CHUNK, D), jnp.float32),
        pltpu.VMEM_SHARED((ACC_ROWS, D), jnp.float32),
        pltpu.SemaphoreType.DMA,
    ],
)
def _sc_segsum(y_hbm, srcp_hbm, dstp_hbm, zrows_hbm, out_hbm,
               src_v, dst_v, rows_v, acc_sh, sem):
    cid = lax.axis_index("c")
    sid = lax.axis_index("s")
    wid = cid * NUM_SUBCORES + sid
    zbase = sid * ZROWS
    pltpu.sync_copy(zrows_hbm, acc_sh.at[pl.ds(zbase, ZROWS)])
    pltpu.sync_copy(srcp_hbm.at[wid], src_v)
    pltpu.sync_copy(dstp_hbm.at[wid], dst_v)
    plsc.subcore_barrier()

    def body(j, carry):
        pltpu.async_copy(y_hbm.at[src_v.at[j]], rows_v, sem).wait()
        pltpu.sync_copy(rows_v, acc_sh.at[dst_v.at[j]], add=True)
        return carry

    lax.fori_loop(0, CHUNKS_PER_TILE, body, 0)

    plsc.subcore_barrier()
    pltpu.sync_copy(acc_sh.at[pl.ds(zbase, ZROWS)],
                    out_hbm.at[cid, pl.ds(zbase, ZROWS)])


_R = 1000


def _tc_scale_matmul_body(degp_ref, x_ref, w_ref, o_ref):
    dis = lax.rsqrt(degp_ref[0] + degp_ref[1] + 1.0)
    o_ref[...] = dis * jnp.dot(x_ref[...], w_ref[...],
                               preferred_element_type=jnp.float32)


def _tc_combine_relu_matmul_body(acc_ref, y_ref, degp_ref, b_ref, w_ref, o_ref):
    dis = lax.rsqrt(degp_ref[0] + degp_ref[1] + 1.0)
    h = dis * (acc_ref[0] + acc_ref[1] + y_ref[...]) + b_ref[...]
    h = jnp.maximum(h, 0.0)
    o_ref[...] = dis * jnp.dot(h, w_ref[...],
                               preferred_element_type=jnp.float32)


def _tc_combine_body(acc_ref, y_ref, degp_ref, b_ref, o_ref):
    dis = lax.rsqrt(degp_ref[0] + degp_ref[1] + 1.0)
    o_ref[...] = dis * (acc_ref[0] + acc_ref[1] + y_ref[...]) + b_ref[...]


_DEG_SPEC = pl.BlockSpec((NUM_CORES, _R, 1), lambda i: (0, i, 0))
_ROW_SPEC = pl.BlockSpec((_R, D), lambda i: (i, 0))
_ACC_SPEC = pl.BlockSpec((NUM_CORES, _R, D), lambda i: (0, i, 0))
_W_SPEC = pl.BlockSpec((D, D), lambda i: (0, 0))
_B_SPEC = pl.BlockSpec((1, D), lambda i: (0, 0))
_OUT_ROWS = jax.ShapeDtypeStruct((N_NODES, D), jnp.float32)


def _tc_scale_matmul(degp, x, w):
    return pl.pallas_call(
        _tc_scale_matmul_body,
        grid=(N_NODES // _R,),
        in_specs=[_DEG_SPEC, _ROW_SPEC, _W_SPEC],
        out_specs=_ROW_SPEC,
        out_shape=_OUT_ROWS,
    )(degp, x, w)


def _tc_combine_relu_matmul(acc, y, degp, b, w):
    return pl.pallas_call(
        _tc_combine_relu_matmul_body,
        grid=(N_NODES // _R,),
        in_specs=[_ACC_SPEC, _ROW_SPEC, _DEG_SPEC, _B_SPEC, _W_SPEC],
        out_specs=_ROW_SPEC,
        out_shape=_OUT_ROWS,
    )(acc, y, degp, b, w)


def _tc_combine(acc, y, degp, b):
    return pl.pallas_call(
        _tc_combine_body,
        grid=(N_NODES // _R,),
        in_specs=[_ACC_SPEC, _ROW_SPEC, _DEG_SPEC, _B_SPEC],
        out_specs=_ROW_SPEC,
        out_shape=_OUT_ROWS,
    )(acc, y, degp, b)


def kernel(x, train_pos_edge_index, W1, b1, W2, b2):
    src = train_pos_edge_index[0].astype(jnp.int32)
    dst = train_pos_edge_index[1].astype(jnp.int32)
    npad = E_PAD - N_EDGES
    srcp = jnp.concatenate([src, jnp.zeros((npad,), jnp.int32)])
    dst_sink = N_NODES + (jnp.arange(npad, dtype=jnp.int32)
                          % (ACC_ROWS - N_NODES))
    dstp = jnp.concatenate([dst, dst_sink])
    srcp = srcp.reshape(NUM_TILES, CHUNKS_PER_TILE, CHUNK)
    dstp = dstp.reshape(NUM_TILES, CHUNKS_PER_TILE, CHUNK)

    zrows = jnp.zeros((ZROWS, D), jnp.float32)
    ones_rows = jnp.ones((CHUNK, D), jnp.float32)
    b1r = b1.reshape(1, D)
    b2r = b2.reshape(1, D)

    degp = _sc_degree(dstp, zrows, ones_rows)[:, :, :1]
    y1 = _tc_scale_matmul(degp, x, W1)
    acc1 = _sc_segsum(y1, srcp, dstp, zrows)
    y2 = _tc_combine_relu_matmul(acc1, y1, degp, b1r, W2)
    acc2 = _sc_segsum(y2, srcp, dstp, zrows)
    z = _tc_combine(acc2, y2, degp, b2r)
    return z


# speedup vs baseline: 7.9901x; 1.0599x over previous
import functools

import jax
import jax.numpy as jnp
from jax import lax
from jax.experimental import pallas as pl
from jax.experimental.pallas import tpu as pltpu
from jax.experimental.pallas import tpu_sc as plsc

N_NODES = 10000
N_EDGES = 320000
D = 128

NUM_CORES = 2
NUM_SUBCORES = 16
NUM_TILES = NUM_CORES * NUM_SUBCORES

CHUNK = 128
BC = 16
BLOCKS = 5
CHUNKS_PER_TILE = BC * BLOCKS
EDGES_PER_TILE = CHUNK * CHUNKS_PER_TILE
E_PAD = EDGES_PER_TILE * NUM_TILES
ACC_ROWS = 10112
ZROWS = ACC_ROWS // NUM_SUBCORES
DEG_W = 16

_mesh = plsc.VectorSubcoreMesh(core_axis_name="c", subcore_axis_name="s")


@functools.partial(
    pl.kernel,
    out_type=jax.ShapeDtypeStruct((NUM_CORES, ACC_ROWS, D), jnp.float32),
    mesh=_mesh,
    scratch_types=[
        pltpu.VMEM((CHUNKS_PER_TILE, CHUNK), jnp.int32),
        pltpu.VMEM((CHUNK, D), jnp.float32),
        pltpu.VMEM_SHARED((ACC_ROWS, D), jnp.float32),
    ],
)
def _sc_degree(dstp_hbm, zrows_hbm, ones_hbm, out_hbm, dst_v, ones_v, deg_sh):
    cid = lax.axis_index("c")
    sid = lax.axis_index("s")
    wid = cid * NUM_SUBCORES + sid
    zbase = sid * ZROWS
    pltpu.sync_copy(zrows_hbm, deg_sh.at[pl.ds(zbase, ZROWS)])
    pltpu.sync_copy(ones_hbm, ones_v)
    pltpu.sync_copy(dstp_hbm.at[wid], dst_v)
    plsc.subcore_barrier()

    def body(j, carry):
        pltpu.sync_copy(ones_v, deg_sh.at[dst_v.at[j]], add=True)
        return carry

    lax.fori_loop(0, CHUNKS_PER_TILE, body, 0)
    plsc.subcore_barrier()
    pltpu.sync_copy(deg_sh.at[pl.ds(zbase, ZROWS)],
                    out_hbm.at[cid, pl.ds(zbase, ZROWS)])


@functools.partial(
    pl.kernel,
    out_type=jax.ShapeDtypeStruct((NUM_CORES, ACC_ROWS, D), jnp.float32),
    mesh=_mesh,
    scratch_types=[
        pltpu.VMEM((BC, CHUNK), jnp.int32),
        pltpu.VMEM((BC, CHUNK), jnp.int32),
        pltpu.VMEM((BC, CHUNK), jnp.int32),
        pltpu.VMEM((BC, CHUNK), jnp.int32),
        pltpu.VMEM((CHUNK, D), jnp.float32),
        pltpu.VMEM((CHUNK, D), jnp.float32),
        pltpu.VMEM_SHARED((ACC_ROWS, D), jnp.float32),
        pltpu.SemaphoreType.DMA,
        pltpu.SemaphoreType.DMA,
        pltpu.SemaphoreType.DMA,
        pltpu.SemaphoreType.DMA,
        pltpu.SemaphoreType.DMA,
    ],
)
def _sc_segsum(y_hbm, srcp_hbm, dstp_hbm, zrows_hbm, out_hbm,
               src_e, src_o, dst_e, dst_o, rows_e, rows_o, acc_sh,
               sem_idx, sem_ge, sem_go, sem_se, sem_so):
    cid = lax.axis_index("c")
    sid = lax.axis_index("s")
    wid = cid * NUM_SUBCORES + sid
    zbase = sid * ZROWS
    pltpu.sync_copy(zrows_hbm, acc_sh.at[pl.ds(zbase, ZROWS)])
    pltpu.sync_copy(srcp_hbm.at[wid, pl.ds(0, BC)], src_e)
    pltpu.sync_copy(dstp_hbm.at[wid, pl.ds(0, BC)], dst_e)
    plsc.subcore_barrier()

    for b in range(BLOCKS):
        src_blk, dst_blk = (src_e, dst_e) if b % 2 == 0 else (src_o, dst_o)
        src_nxt, dst_nxt = (src_o, dst_o) if b % 2 == 0 else (src_e, dst_e)
        if b + 1 < BLOCKS:
            pltpu.async_copy(srcp_hbm.at[wid, pl.ds((b + 1) * BC, BC)],
                             src_nxt, sem_idx)
            pltpu.async_copy(dstp_hbm.at[wid, pl.ds((b + 1) * BC, BC)],
                             dst_nxt, sem_idx)
        pltpu.async_copy(y_hbm.at[src_blk.at[0]], rows_e, sem_ge)

        def body(j, carry):
            @pl.when(j % 2 == 0)
            def _():
                pltpu.make_async_copy(y_hbm.at[src_blk.at[j]], rows_e,
                                      sem_ge).wait()

                @pl.when(j + 1 < BC)
                def _():
                    @pl.when(j > 0)
                    def _():
                        pltpu.make_async_copy(
                            rows_o, acc_sh.at[dst_blk.at[j - 1]],
                            sem_so).wait()
                    pltpu.async_copy(y_hbm.at[src_blk.at[j + 1]], rows_o,
                                     sem_go)
                pltpu.async_copy(rows_e, acc_sh.at[dst_blk.at[j]], sem_se,
                                 add=True)

            @pl.when(j % 2 == 1)
            def _():
                pltpu.make_async_copy(y_hbm.at[src_blk.at[j]], rows_o,
                                      sem_go).wait()

                @pl.when(j + 1 < BC)
                def _():
                    pltpu.make_async_copy(rows_e, acc_sh.at[dst_blk.at[j - 1]],
                                          sem_se).wait()
                    pltpu.async_copy(y_hbm.at[src_blk.at[j + 1]], rows_e,
                                     sem_ge)
                pltpu.async_copy(rows_o, acc_sh.at[dst_blk.at[j]], sem_so,
                                 add=True)

            return carry

        lax.fori_loop(0, BC, body, 0)
        pltpu.make_async_copy(rows_e, acc_sh.at[dst_blk.at[BC - 2]],
                              sem_se).wait()
        pltpu.make_async_copy(rows_o, acc_sh.at[dst_blk.at[BC - 1]],
                              sem_so).wait()
        if b + 1 < BLOCKS:
            pltpu.make_async_copy(srcp_hbm.at[wid, pl.ds((b + 1) * BC, BC)],
                                  src_nxt, sem_idx).wait()
            pltpu.make_async_copy(dstp_hbm.at[wid, pl.ds((b + 1) * BC, BC)],
                                  dst_nxt, sem_idx).wait()

    plsc.subcore_barrier()
    pltpu.sync_copy(acc_sh.at[pl.ds(zbase, ZROWS)],
                    out_hbm.at[cid, pl.ds(zbase, ZROWS)])


_R = 1000


def _tc_scale_matmul_body(degp_ref, x_ref, w_ref, o_ref):
    dis = lax.rsqrt(degp_ref[0] + degp_ref[1] + 1.0)
    o_ref[...] = dis * jnp.dot(x_ref[...], w_ref[...],
                               preferred_element_type=jnp.float32)


def _tc_combine_relu_matmul_body(acc_ref, y_ref, degp_ref, b_ref, w_ref, o_ref):
    dis = lax.rsqrt(degp_ref[0] + degp_ref[1] + 1.0)
    h = dis * (acc_ref[0] + acc_ref[1] + y_ref[...]) + b_ref[...]
    h = jnp.maximum(h, 0.0)
    o_ref[...] = dis * jnp.dot(h, w_ref[...],
                               preferred_element_type=jnp.float32)


def _tc_combine_body(acc_ref, y_ref, degp_ref, b_ref, o_ref):
    dis = lax.rsqrt(degp_ref[0] + degp_ref[1] + 1.0)
    o_ref[...] = dis * (acc_ref[0] + acc_ref[1] + y_ref[...]) + b_ref[...]


_DEG_SPEC = pl.BlockSpec((NUM_CORES, _R, 1), lambda i: (0, i, 0))
_ROW_SPEC = pl.BlockSpec((_R, D), lambda i: (i, 0))
_ACC_SPEC = pl.BlockSpec((NUM_CORES, _R, D), lambda i: (0, i, 0))
_W_SPEC = pl.BlockSpec((D, D), lambda i: (0, 0))
_B_SPEC = pl.BlockSpec((1, D), lambda i: (0, 0))
_OUT_ROWS = jax.ShapeDtypeStruct((N_NODES, D), jnp.float32)


def _tc_scale_matmul(degp, x, w):
    return pl.pallas_call(
        _tc_scale_matmul_body,
        grid=(N_NODES // _R,),
        in_specs=[_DEG_SPEC, _ROW_SPEC, _W_SPEC],
        out_specs=_ROW_SPEC,
        out_shape=_OUT_ROWS,
    )(degp, x, w)


def _tc_combine_relu_matmul(acc, y, degp, b, w):
    return pl.pallas_call(
        _tc_combine_relu_matmul_body,
        grid=(N_NODES // _R,),
        in_specs=[_ACC_SPEC, _ROW_SPEC, _DEG_SPEC, _B_SPEC, _W_SPEC],
        out_specs=_ROW_SPEC,
        out_shape=_OUT_ROWS,
    )(acc, y, degp, b, w)


def _tc_combine(acc, y, degp, b):
    return pl.pallas_call(
        _tc_combine_body,
        grid=(N_NODES // _R,),
        in_specs=[_ACC_SPEC, _ROW_SPEC, _DEG_SPEC, _B_SPEC],
        out_specs=_ROW_SPEC,
        out_shape=_OUT_ROWS,
    )(acc, y, degp, b)


def kernel(x, train_pos_edge_index, W1, b1, W2, b2):
    src = train_pos_edge_index[0].astype(jnp.int32)
    dst = train_pos_edge_index[1].astype(jnp.int32)
    npad = E_PAD - N_EDGES
    srcp = jnp.concatenate([src, jnp.zeros((npad,), jnp.int32)])
    dst_sink = N_NODES + (jnp.arange(npad, dtype=jnp.int32)
                          % (ACC_ROWS - N_NODES))
    dstp = jnp.concatenate([dst, dst_sink])
    srcp = srcp.reshape(NUM_TILES, CHUNKS_PER_TILE, CHUNK)
    dstp = dstp.reshape(NUM_TILES, CHUNKS_PER_TILE, CHUNK)

    zrows = jnp.zeros((ZROWS, D), jnp.float32)
    ones_rows = jnp.ones((CHUNK, D), jnp.float32)
    b1r = b1.reshape(1, D)
    b2r = b2.reshape(1, D)

    degp = _sc_degree(dstp, zrows, ones_rows)[:, :, :1]
    y1 = _tc_scale_matmul(degp, x, W1)
    acc1 = _sc_segsum(y1, srcp, dstp, zrows)
    y2 = _tc_combine_relu_matmul(acc1, y1, degp, b1r, W2)
    acc2 = _sc_segsum(y2, srcp, dstp, zrows)
    z = _tc_combine(acc2, y2, degp, b2r)
    return z
